# baseline (speedup 1.0000x reference)
import functools

import jax
import jax.numpy as jnp
from jax import lax
from jax.experimental import pallas as pl
from jax.experimental.pallas import tpu as pltpu
from jax.experimental.pallas import tpu_sc as plsc

TOPK_K = 128
LANES = 16
N_IMG = 64
N_CHAN = 384
SPAT = 576
ROWS2 = N_IMG * SPAT
CB = 64
NGB = CB // LANES
CBLOCKS = N_CHAN // CB
N_WORKERS = 32
TASKS_PER_W = (N_IMG * CBLOCKS) // N_WORKERS
INT_MIN32 = -(2**31)


def _srl(x, n):
    return lax.shift_right_logical(x, n)


def _sc_body(x_hbm, o_hbm, braw, kbuf, hist):
    c = lax.axis_index("c")
    s = lax.axis_index("s")
    wid = s * 2 + c

    zeros = jnp.zeros((LANES,), jnp.int32)
    ones = jnp.full((LANES,), 1, jnp.int32)
    four = jnp.full((LANES,), 4, jnp.int32)
    cols = [lax.iota(jnp.int32, LANES) + g * LANES for g in range(NGB)]

    def task_body(ti, carry):
        t = wid * TASKS_PER_W + ti
        n = t // CBLOCKS
        cb = t - n * CBLOCKS
        r0 = n * SPAT
        c0 = cb * CB
        pltpu.sync_copy(
            x_hbm.at[pl.ds(r0, SPAT), pl.ds(c0, CB)], braw
        )

        def key_body(sp, kc):
            for g in range(NGB):
                b = braw[sp, pl.ds(g * LANES, LANES)]
                kbuf[sp, pl.ds(g * LANES, LANES)] = b ^ (
                    (b >> 31) | INT_MIN32
                )
            return kc

        lax.fori_loop(0, SPAT, key_body, 0)

        for d in range(16):
            for g in range(NGB):
                hist[d, pl.ds(g * LANES, LANES)] = zeros

        prefix = [zeros] * NGB
        krem = [jnp.full((LANES,), TOPK_K, jnp.int32)] * NGB
        for p in range(8):
            shift = jnp.full((LANES,), 28 - 4 * p, jnp.int32)
            pfx = prefix

            def hist_body(sp, hc, pfx=pfx, shift=shift):
                for g in range(NGB):
                    k = kbuf[sp, pl.ds(g * LANES, LANES)]
                    hi = _srl(k, shift)
                    dig = hi & jnp.int32(0xF)
                    m = _srl(hi, four) == pfx[g]
                    plsc.addupdate_scatter(
                        hist, [dig, cols[g]], ones, mask=m
                    )
                return hc

            lax.fori_loop(0, SPAT, hist_body, 0)

            new_prefix = []
            new_krem = []
            for g in range(NGB):
                cum = zeros
                found = zeros > ones
                digit = zeros
                above = zeros
                for d in range(15, -1, -1):
                    hd = hist[d, pl.ds(g * LANES, LANES)]
                    hist[d, pl.ds(g * LANES, LANES)] = zeros
                    newcum = cum + hd
                    cross = (newcum >= krem[g]) & (~found)
                    digit = jnp.where(cross, jnp.int32(d), digit)
                    above = jnp.where(cross, cum, above)
                    found = found | cross
                    cum = newcum
                new_krem.append(krem[g] - above)
                new_prefix.append((prefix[g] << four) | digit)
            prefix = new_prefix
            krem = new_krem

        tsig = [prefix[g] ^ INT_MIN32 for g in range(NGB)]

        def apply_body(sp, ac):
            for g in range(NGB):
                k = kbuf[sp, pl.ds(g * LANES, LANES)]
                m = (k ^ INT_MIN32) >= tsig[g]
                b = braw[sp, pl.ds(g * LANES, LANES)]
                braw[sp, pl.ds(g * LANES, LANES)] = jnp.where(m, b, zeros)
            return ac

        lax.fori_loop(0, SPAT, apply_body, 0)
        pltpu.sync_copy(
            braw, o_hbm.at[pl.ds(r0, SPAT), pl.ds(c0, CB)]
        )
        return carry

    lax.fori_loop(0, TASKS_PER_W, task_body, 0)


@jax.jit
def _sc_sparsify(xt):
    mesh = plsc.VectorSubcoreMesh(core_axis_name="c", subcore_axis_name="s")
    fn = pl.kernel(
        _sc_body,
        out_type=jax.ShapeDtypeStruct((ROWS2, N_CHAN), jnp.int32),
        mesh=mesh,
        compiler_params=pltpu.CompilerParams(
            needs_layout_passes=False, use_tc_tiling_on_sc=False
        ),
        scratch_types=[
            pltpu.VMEM((SPAT, CB), jnp.int32),
            pltpu.VMEM((SPAT, CB), jnp.int32),
            pltpu.VMEM((16, CB), jnp.int32),
        ],
    )
    return fn(xt)


def kernel(x):
    n, c, h, w = x.shape
    xr = lax.bitcast_convert_type(x, jnp.int32)
    xt = jnp.transpose(xr, (0, 2, 3, 1)).reshape(n * h * w, c)
    out = _sc_sparsify(xt)
    out = jnp.transpose(out.reshape(n, h, w, c), (0, 3, 1, 2))
    return lax.bitcast_convert_type(out, jnp.float32)


# speedup vs baseline: 1.0488x; 1.0488x over previous
import functools

import jax
import jax.numpy as jnp
from jax import lax
from jax.experimental import pallas as pl
from jax.experimental.pallas import tpu as pltpu
from jax.experimental.pallas import tpu_sc as plsc

TOPK_K = 128
LANES = 16
N_IMG = 64
N_CHAN = 384
SPAT = 576
ROWS2 = N_IMG * SPAT
CB = 64
NGB = CB // LANES
CBLOCKS = N_CHAN // CB
N_WORKERS = 32
TASKS_PER_W = (N_IMG * CBLOCKS) // N_WORKERS
UNROLL = 16
SP_ITERS = SPAT // UNROLL
INT_MIN32 = -(2**31)


def _srl(x, n):
    return lax.shift_right_logical(x, n)


def _sc_body(x_hbm, o_hbm, braw, kbuf, hist):
    c = lax.axis_index("c")
    s = lax.axis_index("s")
    wid = s * 2 + c

    zeros = jnp.zeros((LANES,), jnp.int32)
    ones = jnp.full((LANES,), 1, jnp.int32)
    four = jnp.full((LANES,), 4, jnp.int32)
    cols = [lax.iota(jnp.int32, LANES) + g * LANES for g in range(NGB)]

    def task_body(ti, carry):
        t = wid * TASKS_PER_W + ti
        n = t // CBLOCKS
        cb = t - n * CBLOCKS
        r0 = n * SPAT
        c0 = cb * CB
        pltpu.sync_copy(
            x_hbm.at[pl.ds(r0, SPAT), pl.ds(c0, CB)], braw
        )

        def key_body(it, kc):
            sp0 = it * UNROLL
            for u in range(UNROLL):
                for g in range(NGB):
                    b = braw[sp0 + u, pl.ds(g * LANES, LANES)]
                    kbuf[sp0 + u, pl.ds(g * LANES, LANES)] = b ^ (
                        (b >> 31) | INT_MIN32
                    )
            return kc

        lax.fori_loop(0, SP_ITERS, key_body, 0)

        for d in range(16):
            for g in range(NGB):
                hist[d, pl.ds(g * LANES, LANES)] = zeros

        prefix = [zeros] * NGB
        krem = [jnp.full((LANES,), TOPK_K, jnp.int32)] * NGB
        for p in range(8):
            shift = jnp.full((LANES,), 28 - 4 * p, jnp.int32)
            pfx = prefix

            def hist_body(it, hc, pfx=pfx, shift=shift):
                sp0 = it * UNROLL
                for u in range(UNROLL):
                    for g in range(NGB):
                        k = kbuf[sp0 + u, pl.ds(g * LANES, LANES)]
                        hi = _srl(k, shift)
                        dig = hi & jnp.int32(0xF)
                        m = _srl(hi, four) == pfx[g]
                        plsc.addupdate_scatter(
                            hist, [dig, cols[g]], ones, mask=m
                        )
                return hc

            lax.fori_loop(0, SP_ITERS, hist_body, 0)

            new_prefix = []
            new_krem = []
            for g in range(NGB):

                def scan_body(i, st, g=g, krem_g=krem[g]):
                    cum, found, digit, above = st
                    d = 15 - i
                    hd = hist[d, pl.ds(g * LANES, LANES)]
                    hist[d, pl.ds(g * LANES, LANES)] = zeros
                    newcum = cum + hd
                    cross = (newcum >= krem_g) & (~found)
                    digit = jnp.where(cross, d, digit)
                    above = jnp.where(cross, cum, above)
                    return newcum, found | cross, digit, above

                falses = zeros > ones
                _, _, digit, above = lax.fori_loop(
                    0, 16, scan_body, (zeros, falses, zeros, zeros)
                )
                new_krem.append(krem[g] - above)
                new_prefix.append((prefix[g] << four) | digit)
            prefix = new_prefix
            krem = new_krem

        tsig = [prefix[g] ^ INT_MIN32 for g in range(NGB)]

        def apply_body(it, ac):
            sp0 = it * UNROLL
            for u in range(UNROLL):
                for g in range(NGB):
                    k = kbuf[sp0 + u, pl.ds(g * LANES, LANES)]
                    m = (k ^ INT_MIN32) >= tsig[g]
                    b = braw[sp0 + u, pl.ds(g * LANES, LANES)]
                    braw[sp0 + u, pl.ds(g * LANES, LANES)] = jnp.where(
                        m, b, zeros
                    )
            return ac

        lax.fori_loop(0, SP_ITERS, apply_body, 0)
        pltpu.sync_copy(
            braw, o_hbm.at[pl.ds(r0, SPAT), pl.ds(c0, CB)]
        )
        return carry

    lax.fori_loop(0, TASKS_PER_W, task_body, 0)


@jax.jit
def _sc_sparsify(xt):
    mesh = plsc.VectorSubcoreMesh(core_axis_name="c", subcore_axis_name="s")
    fn = pl.kernel(
        _sc_body,
        out_type=jax.ShapeDtypeStruct((ROWS2, N_CHAN), jnp.int32),
        mesh=mesh,
        compiler_params=pltpu.CompilerParams(
            needs_layout_passes=False, use_tc_tiling_on_sc=False
        ),
        scratch_types=[
            pltpu.VMEM((SPAT, CB), jnp.int32),
            pltpu.VMEM((SPAT, CB), jnp.int32),
            pltpu.VMEM((16, CB), jnp.int32),
        ],
    )
    return fn(xt)


def kernel(x):
    n, c, h, w = x.shape
    xr = lax.bitcast_convert_type(x, jnp.int32)
    xt = jnp.transpose(xr, (0, 2, 3, 1)).reshape(n * h * w, c)
    out = _sc_sparsify(xt)
    out = jnp.transpose(out.reshape(n, h, w, c), (0, 3, 1, 2))
    return lax.bitcast_convert_type(out, jnp.float32)


# speedup vs baseline: 2.6371x; 2.5145x over previous
import functools

import jax
import jax.numpy as jnp
from jax import lax
from jax.experimental import pallas as pl
from jax.experimental.pallas import tpu as pltpu
from jax.experimental.pallas import tpu_sc as plsc

TOPK_K = 128
N_ROWS = 24576
ROW_LEN = 576
LANES = 16
NVEC = ROW_LEN // LANES
N_WORKERS = 32
SC_ROWS = 12288
SC_ROWS_PER_W = SC_ROWS // N_WORKERS
SC_CHUNK = 128
SC_N_CHUNKS = SC_ROWS_PER_W // SC_CHUNK
TC_BLOCK = 256
INT_MIN32 = -(2**31)


def _sc_body(x_hbm, o_hbm, buf):
    c = lax.axis_index("c")
    s = lax.axis_index("s")
    wid = s * 2 + c
    row0 = wid * SC_ROWS_PER_W

    def chunk_body(ci, carry):
        base = row0 + ci * SC_CHUNK
        pltpu.sync_copy(x_hbm.at[pl.ds(base, SC_CHUNK)], buf)

        def row_body(r, rcarry):
            keys = []
            for j in range(NVEC):
                b = buf[r, pl.ds(j * LANES, LANES)]
                skey = b ^ ((b >> 31) & jnp.int32(0x7FFFFFFF))
                keys.append(skey)

            def bit_body(i, tb):
                cand_b = tb | jnp.full((LANES,), 1, jnp.int32) << (31 - i)
                cand = cand_b ^ INT_MIN32
                total = jnp.zeros((LANES,), jnp.int32)
                for kj in keys:
                    total = total + plsc.all_reduce_population_count(
                        kj >= cand
                    )
                return jnp.where(total >= TOPK_K, cand_b, tb)

            tb0 = jnp.zeros((LANES,), jnp.int32)
            tb = lax.fori_loop(0, 32, bit_body, tb0)
            t = tb ^ INT_MIN32
            zero = jnp.zeros((LANES,), jnp.int32)
            for j in range(NVEC):
                bv = buf[r, pl.ds(j * LANES, LANES)]
                buf[r, pl.ds(j * LANES, LANES)] = jnp.where(
                    keys[j] >= t, bv, zero
                )
            return rcarry

        lax.fori_loop(0, SC_CHUNK, row_body, 0)
        pltpu.sync_copy(buf, o_hbm.at[pl.ds(base, SC_CHUNK)])
        return carry

    lax.fori_loop(0, SC_N_CHUNKS, chunk_body, 0)


@jax.jit
def _sc_sparsify(xr):
    mesh = plsc.VectorSubcoreMesh(core_axis_name="c", subcore_axis_name="s")
    fn = pl.kernel(
        _sc_body,
        out_type=jax.ShapeDtypeStruct((SC_ROWS, ROW_LEN), jnp.int32),
        mesh=mesh,
        compiler_params=pltpu.CompilerParams(needs_layout_passes=False),
        scratch_types=[pltpu.VMEM((SC_CHUNK, ROW_LEN), jnp.int32)],
    )
    return fn(xr)


def _tc_body(x_ref, o_ref):
    xb = x_ref[...]
    b = lax.bitcast_convert_type(xb, jnp.int32)
    ub = lax.bitcast_convert_type(xb, jnp.uint32)
    ukey = jnp.where(b < 0, ~ub, ub | jnp.uint32(0x80000000))

    def bit_step(i, t):
        bit = jnp.uint32(31) - i.astype(jnp.uint32)
        cand = t | (jnp.uint32(1) << bit)
        cnt = jnp.sum((ukey >= cand).astype(jnp.int32), axis=1, keepdims=True)
        return jnp.where(cnt >= TOPK_K, cand, t)

    t0 = jnp.zeros((xb.shape[0], 1), jnp.uint32)
    t = lax.fori_loop(0, 32, bit_step, t0)
    o_ref[...] = jnp.where(ukey >= t, xb, 0.0)


def _tc_sparsify(xr):
    rows = xr.shape[0]
    return pl.pallas_call(
        _tc_body,
        grid=(rows // TC_BLOCK,),
        in_specs=[pl.BlockSpec((TC_BLOCK, ROW_LEN), lambda i: (i, 0))],
        out_specs=pl.BlockSpec((TC_BLOCK, ROW_LEN), lambda i: (i, 0)),
        out_shape=jax.ShapeDtypeStruct((rows, ROW_LEN), xr.dtype),
    )(xr)


def kernel(x):
    n, c, h, w = x.shape
    xr = x.reshape(n * c, h * w)
    xi = lax.bitcast_convert_type(xr[:SC_ROWS], jnp.int32)
    sc_out = lax.bitcast_convert_type(_sc_sparsify(xi), jnp.float32)
    tc_out = _tc_sparsify(xr[SC_ROWS:])
    out = jnp.concatenate([sc_out, tc_out], axis=0)
    return out.reshape(n, c, h, w)
